# baseline (speedup 1.0000x reference)
import jax
import jax.numpy as jnp
from jax.experimental import pallas as pl
from jax.experimental.pallas import tpu as pltpu

_NUM_CLASSES = 1000
_NUM_GROUPS = 10
_COST = 0.05
_EPS = 1e-12
_BLOCK_B = 4096


def _body(cls_ref, alpha_ref, mu_ref, post_ref, pred_ref, rej_ref):
    cls = cls_ref[...]
    a = jnp.zeros(cls.shape, jnp.float32)
    m = jnp.zeros(cls.shape, jnp.float32)
    for g in range(_NUM_GROUPS):
        sel = cls == g
        a = jnp.where(sel, alpha_ref[g], a)
        m = jnp.where(sel, mu_ref[g], m)
    ah = jnp.maximum(a / float(_NUM_GROUPS), _EPS)
    w2 = 1.0 / ah - m

    p = post_ref[...]
    thr = jax.lax.dot_general(
        w2, p, (((0,), (0,)), ((), ())),
        preferred_element_type=jnp.float32,
    )
    rwd = p / ah
    mxd = jnp.max(rwd, axis=0, keepdims=True)
    iota_col = jax.lax.broadcasted_iota(jnp.int32, (_NUM_CLASSES, 1), 0)
    pred_ref[...] = jnp.min(
        jnp.where(rwd == mxd, iota_col, _NUM_CLASSES), axis=0, keepdims=True
    )
    rej_ref[...] = jnp.where(mxd < thr - _COST, 1, 0).astype(jnp.int32)


def kernel(posterior, class_to_group, alpha_group, mu_group):
    B, C = posterior.shape
    pt = posterior.T
    cls2 = class_to_group.reshape(C, 1)
    grid = (B // _BLOCK_B,)
    pred2, rej2 = pl.pallas_call(
        _body,
        grid=grid,
        in_specs=[
            pl.BlockSpec((C, 1), lambda i: (0, 0)),
            pl.BlockSpec(memory_space=pltpu.SMEM),
            pl.BlockSpec(memory_space=pltpu.SMEM),
            pl.BlockSpec((C, _BLOCK_B), lambda i: (0, i)),
        ],
        out_specs=[
            pl.BlockSpec((1, _BLOCK_B), lambda i: (0, i)),
            pl.BlockSpec((1, _BLOCK_B), lambda i: (0, i)),
        ],
        out_shape=[
            jax.ShapeDtypeStruct((1, B), jnp.int32),
            jax.ShapeDtypeStruct((1, B), jnp.int32),
        ],
        compiler_params=pltpu.CompilerParams(
            dimension_semantics=("parallel",),
        ),
    )(cls2, alpha_group, mu_group, pt)
    return pred2.reshape(B), rej2.reshape(B).astype(bool)


# speedup vs baseline: 1.0742x; 1.0742x over previous
import jax
import jax.numpy as jnp
from jax.experimental import pallas as pl
from jax.experimental.pallas import tpu as pltpu

_NUM_CLASSES = 1000
_NUM_GROUPS = 10
_COST = 0.05
_EPS = 1e-12
_BLOCK_B = 2048


def _body(cls_ref, alpha_ref, mu_ref, post_ref, pred_ref, rej_ref,
          ah_s, w2_s):
    i = pl.program_id(0)

    @pl.when(i == 0)
    def _():
        cls = cls_ref[...]
        a = jnp.zeros(cls.shape, jnp.float32)
        m = jnp.zeros(cls.shape, jnp.float32)
        for g in range(_NUM_GROUPS):
            sel = cls == g
            a = jnp.where(sel, alpha_ref[g], a)
            m = jnp.where(sel, mu_ref[g], m)
        ah = jnp.maximum(a / float(_NUM_GROUPS), _EPS)
        ah_s[...] = ah
        w2_s[...] = 1.0 / ah - m

    ah = ah_s[...]
    p = post_ref[...]
    thr = jax.lax.dot_general(
        w2_s[...], p, (((0,), (0,)), ((), ())),
        preferred_element_type=jnp.float32,
    )
    rwd = p / ah
    mxd = jnp.max(rwd, axis=0, keepdims=True)
    iota_col = jax.lax.broadcasted_iota(jnp.int32, (_NUM_CLASSES, 1), 0)
    pred_ref[...] = jnp.min(
        jnp.where(rwd == mxd, iota_col, _NUM_CLASSES), axis=0, keepdims=True
    )
    rej_ref[...] = jnp.where(mxd < thr - _COST, 1, 0).astype(jnp.int32)


def kernel(posterior, class_to_group, alpha_group, mu_group):
    B, C = posterior.shape
    pt = posterior.T
    cls2 = class_to_group.reshape(C, 1)
    grid = (B // _BLOCK_B,)
    pred2, rej2 = pl.pallas_call(
        _body,
        grid=grid,
        in_specs=[
            pl.BlockSpec((C, 1), lambda i: (0, 0)),
            pl.BlockSpec(memory_space=pltpu.SMEM),
            pl.BlockSpec(memory_space=pltpu.SMEM),
            pl.BlockSpec((C, _BLOCK_B), lambda i: (0, i)),
        ],
        out_specs=[
            pl.BlockSpec((1, _BLOCK_B), lambda i: (0, i)),
            pl.BlockSpec((1, _BLOCK_B), lambda i: (0, i)),
        ],
        out_shape=[
            jax.ShapeDtypeStruct((1, B), jnp.int32),
            jax.ShapeDtypeStruct((1, B), jnp.int32),
        ],
        scratch_shapes=[
            pltpu.VMEM((_NUM_CLASSES, 1), jnp.float32),
            pltpu.VMEM((_NUM_CLASSES, 1), jnp.float32),
        ],
        compiler_params=pltpu.CompilerParams(
            dimension_semantics=("arbitrary",),
        ),
    )(cls2, alpha_group, mu_group, pt)
    return pred2.reshape(B), rej2.reshape(B).astype(bool)
